# baseline (speedup 1.0000x reference)
import jax
import jax.numpy as jnp
import numpy as np
from jax import lax
from jax.experimental import pallas as pl
from jax.experimental.pallas import tpu as pltpu
from jax.experimental.pallas import tpu_sc as plsc

NN = 10000
EE = 320000
DI = 128
DH = 128
DO = 40
DOP = 64

NC = 2
NS = 16
NW = NC * NS
N_PAD = 10240
E_PAD = 327680
CH = 128
CWE = E_PAD // NW // CH
CWC = E_PAD // NS // CH
RPS = N_PAD // NS
_NBUF = 4

_mesh = plsc.VectorSubcoreMesh(core_axis_name="c", subcore_axis_name="s")
_SC_LINEAR = pltpu.CompilerParams(use_tc_tiling_on_sc=False)


def _deg_body(dst_hbm, degp_hbm, dst_idx, ones_v, zbuf, acc, sem):
    c = lax.axis_index("c")
    s = lax.axis_index("s")
    wid = s * NC + c
    pltpu.sync_copy(dst_hbm.at[wid], dst_idx)
    for k in range(CH // 16):
        ones_v[pl.ds(16 * k, 16)] = jnp.ones((16,), jnp.float32)
    for k in range(RPS // 16):
        zbuf[pl.ds(16 * k, 16)] = jnp.zeros((16,), jnp.float32)
    pltpu.sync_copy(zbuf, acc.at[pl.ds(s * RPS, RPS)])
    plsc.subcore_barrier()

    def body(j, carry):
        pltpu.sync_copy(ones_v, acc.at[dst_idx.at[j]], add=True)
        return carry

    lax.fori_loop(0, CWE, body, 0)
    plsc.subcore_barrier()
    pltpu.sync_copy(acc.at[pl.ds(s * RPS, RPS)],
                    degp_hbm.at[pl.ds(c * N_PAD + s * RPS, RPS)])
    del sem


def _sc_degree(dst_sh):
    return pl.kernel(
        _deg_body,
        out_type=jax.ShapeDtypeStruct((NC * N_PAD,), jnp.float32),
        mesh=_mesh,
        scratch_types=[
            pltpu.VMEM((CWE, CH), jnp.int32),
            pltpu.VMEM((CH,), jnp.float32),
            pltpu.VMEM((RPS,), jnp.float32),
            pltpu.VMEM_SHARED((N_PAD,), jnp.float32),
            pltpu.SemaphoreType.DMA,
        ],
        compiler_params=_SC_LINEAR,
    )(dst_sh)


def _agg_pipeline(table_hbm, src_idx, dst_idx, rows, acc, sems, n_chunks):
    for k in range(_NBUF - 1):
        pltpu.async_copy(table_hbm.at[src_idx.at[k]], rows.at[k], sems[k])

    def body(jj, carry):
        for k in range(_NBUF):
            j = _NBUF * jj + k
            pltpu.make_async_copy(
                table_hbm.at[src_idx.at[j]], rows.at[k], sems[k]).wait()
            pltpu.sync_copy(rows.at[k], acc.at[dst_idx.at[j]], add=True)
            kn = (k + _NBUF - 1) % _NBUF

            @pl.when(j + _NBUF - 1 < n_chunks)
            def _():
                pltpu.async_copy(table_hbm.at[src_idx.at[j + _NBUF - 1]],
                                 rows.at[kn], sems[kn])
        return carry

    lax.fori_loop(0, n_chunks // _NBUF, body, 0)


DOH = DOP // 2


def _make_agg_c_body(dc):
    def _agg_c_body(table_hbm, src_hbm, dst_hbm, out_hbm,
                    src_idx, dst_idx, rows, zbuf, acc, *sems):
        c = lax.axis_index("c")
        s = lax.axis_index("s")
        pltpu.sync_copy(src_hbm.at[c, s], src_idx)
        pltpu.sync_copy(dst_hbm.at[s], dst_idx)
        for r in range(16):
            for k in range(dc // 16):
                zbuf[r, pl.ds(16 * k, 16)] = jnp.zeros((16,), jnp.float32)
        for i in range(RPS // 16):
            pltpu.sync_copy(zbuf, acc.at[pl.ds(s * RPS + i * 16, 16)])
        plsc.subcore_barrier()
        _agg_pipeline(table_hbm, src_idx, dst_idx, rows, acc, sems, CWC)
        plsc.subcore_barrier()
        pltpu.sync_copy(acc.at[pl.ds(s * RPS, RPS)],
                        out_hbm.at[pl.ds(c * N_PAD + s * RPS, RPS)])

    return _agg_c_body


def _sc_aggregate_cols(table_flat, src_sh, dst_sh, dc):
    return pl.kernel(
        _make_agg_c_body(dc),
        out_type=jax.ShapeDtypeStruct((NC * N_PAD, dc), jnp.float32),
        mesh=_mesh,
        scratch_types=[
            pltpu.VMEM((CWC, CH), jnp.int32),
            pltpu.VMEM((CWC, CH), jnp.int32),
            pltpu.VMEM((_NBUF, CH, dc), jnp.float32),
            pltpu.VMEM((16, dc), jnp.float32),
            pltpu.VMEM_SHARED((N_PAD, dc), jnp.float32),
        ] + [pltpu.SemaphoreType.DMA] * _NBUF,
        compiler_params=_SC_LINEAR,
    )(table_flat, src_sh, dst_sh)


_GRID = N_PAD // 128
DHH = DH // 2


def _diag(v_row):
    ir = lax.broadcasted_iota(jnp.int32, (128, 128), 0)
    ic = lax.broadcasted_iota(jnp.int32, (128, 128), 1)
    return jnp.where(ir == ic, jnp.broadcast_to(v_row, (128, 128)), 0.0)


def _tc1_body(degp_ref, x_ref, w1_ref, t1_ref, dinv_ref):
    deg = degp_ref[0, 0] + degp_ref[1, 0] + 1.0
    dinv = lax.rsqrt(jnp.maximum(deg, 1.0))
    dmat = _diag(dinv)
    y = jnp.dot(x_ref[...], w1_ref[...], preferred_element_type=jnp.float32)
    y = jnp.dot(dmat, y, preferred_element_type=jnp.float32)
    t1_ref[0] = y[:, :DHH]
    t1_ref[1] = y[:, DHH:]
    dinv_ref[0] = dinv


def _tc_scale_in(degp, x_pad, w1):
    return pl.pallas_call(
        _tc1_body,
        grid=(_GRID,),
        in_specs=[
            pl.BlockSpec((NC, 1, 1, 128), lambda i: (0, i, 0, 0)),
            pl.BlockSpec((128, DI), lambda i: (i, 0)),
            pl.BlockSpec((DI, DH), lambda i: (0, 0)),
        ],
        out_specs=[
            pl.BlockSpec((NC, 128, DHH), lambda i: (0, i, 0)),
            pl.BlockSpec((1, 1, 128), lambda i: (i, 0, 0)),
        ],
        out_shape=[
            jax.ShapeDtypeStruct((NC, N_PAD, DHH), jnp.float32),
            jax.ShapeDtypeStruct((_GRID, 1, 128), jnp.float32),
        ],
    )(degp, x_pad, w1)


def _tc2_body(pp_ref, t1_ref, dinv_ref, b1_ref, w2_ref, t2_ref):
    dmat = _diag(dinv_ref[0])
    agg = jnp.concatenate([pp_ref[0], pp_ref[1]], axis=-1)
    t1 = jnp.concatenate([t1_ref[0], t1_ref[1]], axis=-1)
    pre = jnp.dot(dmat, agg + t1, preferred_element_type=jnp.float32)
    h = jnp.maximum(pre + b1_ref[...], 0.0)
    y2 = jnp.dot(h, w2_ref[...], preferred_element_type=jnp.float32)
    y2 = jnp.dot(dmat, y2, preferred_element_type=jnp.float32)
    t2_ref[0] = y2[:, :DOH]
    t2_ref[1] = y2[:, DOH:]


def _tc_hidden(pp1, table1, dinv, b1r, w2p):
    return pl.pallas_call(
        _tc2_body,
        grid=(_GRID,),
        in_specs=[
            pl.BlockSpec((NC, 128, DHH), lambda i: (0, i, 0)),
            pl.BlockSpec((NC, 128, DHH), lambda i: (0, i, 0)),
            pl.BlockSpec((1, 1, 128), lambda i: (i, 0, 0)),
            pl.BlockSpec((1, DH), lambda i: (0, 0)),
            pl.BlockSpec((DH, DOP), lambda i: (0, 0)),
        ],
        out_specs=pl.BlockSpec((NC, 128, DOH), lambda i: (0, i, 0)),
        out_shape=jax.ShapeDtypeStruct((NC, N_PAD, DOH), jnp.float32),
    )(pp1, table1, dinv, b1r, w2p)


def _tc3_body(pp_ref, t2_ref, dinv_ref, b2_ref, out_ref):
    dmat = _diag(dinv_ref[0])
    agg = jnp.concatenate([pp_ref[0], pp_ref[1]], axis=-1)
    t2 = jnp.concatenate([t2_ref[0], t2_ref[1]], axis=-1)
    z = jnp.dot(dmat, agg + t2, preferred_element_type=jnp.float32)
    z = z + b2_ref[...]
    m = jnp.max(z, axis=1, keepdims=True)
    e = jnp.exp(z - m)
    ssum = jnp.sum(e, axis=1, keepdims=True)
    out_ref[...] = (z - m) - jnp.log(ssum)


def _tc_logsoftmax(pp2, table2, dinv, b2r):
    return pl.pallas_call(
        _tc3_body,
        grid=(_GRID,),
        in_specs=[
            pl.BlockSpec((NC, 128, DOH), lambda i: (0, i, 0)),
            pl.BlockSpec((NC, 128, DOH), lambda i: (0, i, 0)),
            pl.BlockSpec((1, 1, 128), lambda i: (i, 0, 0)),
            pl.BlockSpec((1, DOP), lambda i: (0, 0)),
        ],
        out_specs=pl.BlockSpec((128, DOP), lambda i: (i, 0)),
        out_shape=jax.ShapeDtypeStruct((N_PAD, DOP), jnp.float32),
    )(pp2, table2, dinv, b2r)


def kernel(x, edge_index, W1, b1, W2, b2):
    pad_n = E_PAD - EE
    pad_rows = jnp.asarray(NN + (np.arange(pad_n, dtype=np.int32)
                                 % (N_PAD - NN)))
    src_flat = jnp.concatenate([edge_index[0], pad_rows])
    dst_flat = jnp.concatenate([edge_index[1], pad_rows])
    dst_e = dst_flat.reshape(NW, CWE, CH)
    src_c = jnp.stack([src_flat, src_flat + N_PAD]).reshape(NC, NS, CWC, CH)
    dst_c = dst_flat.reshape(NS, CWC, CH)
    x_pad = jnp.zeros((N_PAD, DI), jnp.float32).at[:NN].set(x)
    w2p = jnp.zeros((DH, DOP), jnp.float32).at[:, :DO].set(W2)
    b1r = b1.reshape(1, DH)
    b2r = jnp.full((1, DOP), -1e30, jnp.float32).at[0, :DO].set(b2)

    degp = _sc_degree(dst_e).reshape(NC, _GRID, 1, 128)
    table1, dinv = _tc_scale_in(degp, x_pad, W1)
    pp1 = _sc_aggregate_cols(table1.reshape(NC * N_PAD, DHH), src_c, dst_c,
                             DHH).reshape(NC, N_PAD, DHH)
    table2 = _tc_hidden(pp1, table1, dinv, b1r, w2p)
    pp2 = _sc_aggregate_cols(table2.reshape(NC * N_PAD, DOH), src_c, dst_c,
                             DOH).reshape(NC, N_PAD, DOH)
    out = _tc_logsoftmax(pp2, table2, dinv, b2r)
    return out[:NN, :DO]


# speedup vs baseline: 1.4418x; 1.4418x over previous
import jax
import jax.numpy as jnp
import numpy as np
from jax import lax
from jax.experimental import pallas as pl
from jax.experimental.pallas import tpu as pltpu
from jax.experimental.pallas import tpu_sc as plsc

NN = 10000
EE = 320000
DI = 128
DH = 128
DO = 40
DOP = 64

NC = 2
NS = 16
NW = NC * NS
N_PAD = 10240
E_PAD = 327680
CH = 128
CWE = E_PAD // NW // CH
CWC = E_PAD // NS // CH
RPS = N_PAD // NS
_NBUF = 4

_mesh = plsc.VectorSubcoreMesh(core_axis_name="c", subcore_axis_name="s")
_SC_LINEAR = pltpu.CompilerParams(use_tc_tiling_on_sc=False)


def _deg_body(dst_hbm, degp_hbm, dst_idx, ones_v, zbuf, acc, sem):
    c = lax.axis_index("c")
    s = lax.axis_index("s")
    wid = s * NC + c
    pltpu.sync_copy(dst_hbm.at[wid], dst_idx)
    for k in range(CH // 16):
        ones_v[pl.ds(16 * k, 16)] = jnp.ones((16,), jnp.float32)
    for k in range(RPS // 16):
        zbuf[pl.ds(16 * k, 16)] = jnp.zeros((16,), jnp.float32)
    pltpu.sync_copy(zbuf, acc.at[pl.ds(s * RPS, RPS)])
    plsc.subcore_barrier()

    def body(j, carry):
        pltpu.sync_copy(ones_v, acc.at[dst_idx.at[j]], add=True)
        return carry

    lax.fori_loop(0, CWE, body, 0)
    plsc.subcore_barrier()
    pltpu.sync_copy(acc.at[pl.ds(s * RPS, RPS)],
                    degp_hbm.at[pl.ds(c * N_PAD + s * RPS, RPS)])
    del sem


def _sc_degree(dst_sh):
    return pl.kernel(
        _deg_body,
        out_type=jax.ShapeDtypeStruct((NC * N_PAD,), jnp.float32),
        mesh=_mesh,
        scratch_types=[
            pltpu.VMEM((CWE, CH), jnp.int32),
            pltpu.VMEM((CH,), jnp.float32),
            pltpu.VMEM((RPS,), jnp.float32),
            pltpu.VMEM_SHARED((N_PAD,), jnp.float32),
            pltpu.SemaphoreType.DMA,
        ],
        compiler_params=_SC_LINEAR,
    )(dst_sh)


def _agg_pipeline(table_hbm, src_idx, dst_idx, rows, acc, sems, n_chunks):
    for k in range(_NBUF - 1):
        pltpu.async_copy(table_hbm.at[src_idx.at[k]], rows.at[k], sems[k])

    def body(jj, carry):
        for k in range(_NBUF):
            j = _NBUF * jj + k
            pltpu.make_async_copy(
                table_hbm.at[src_idx.at[j]], rows.at[k], sems[k]).wait()
            pltpu.sync_copy(rows.at[k], acc.at[dst_idx.at[j]], add=True)
            kn = (k + _NBUF - 1) % _NBUF

            @pl.when(j + _NBUF - 1 < n_chunks)
            def _():
                pltpu.async_copy(table_hbm.at[src_idx.at[j + _NBUF - 1]],
                                 rows.at[kn], sems[kn])
        return carry

    lax.fori_loop(0, n_chunks // _NBUF, body, 0)


DOH = DOP // 2


def _make_agg_c_body(dc):
    def _agg_c_body(table_hbm, src_hbm, dst_hbm, out_hbm,
                    src_idx, dst_idx, rows, zbuf, acc, *sems):
        c = lax.axis_index("c")
        s = lax.axis_index("s")
        pltpu.sync_copy(src_hbm.at[c, s], src_idx)
        pltpu.sync_copy(dst_hbm.at[s], dst_idx)
        for r in range(16):
            for k in range(dc // 16):
                zbuf[r, pl.ds(16 * k, 16)] = jnp.zeros((16,), jnp.float32)
        for i in range(RPS // 16):
            pltpu.sync_copy(zbuf, acc.at[pl.ds(s * RPS + i * 16, 16)])
        plsc.subcore_barrier()
        _agg_pipeline(table_hbm, src_idx, dst_idx, rows, acc, sems, CWC)
        plsc.subcore_barrier()
        pltpu.sync_copy(acc.at[pl.ds(s * RPS, RPS)],
                        out_hbm.at[pl.ds(c * N_PAD + s * RPS, RPS)])

    return _agg_c_body


def _sc_aggregate_cols(table_flat, src_sh, dst_sh, dc):
    return pl.kernel(
        _make_agg_c_body(dc),
        out_type=jax.ShapeDtypeStruct((NC * N_PAD, dc), jnp.float32),
        mesh=_mesh,
        scratch_types=[
            pltpu.VMEM((CWC, CH), jnp.int32),
            pltpu.VMEM((CWC, CH), jnp.int32),
            pltpu.VMEM((_NBUF, CH, dc), jnp.float32),
            pltpu.VMEM((16, dc), jnp.float32),
            pltpu.VMEM_SHARED((N_PAD, dc), jnp.float32),
        ] + [pltpu.SemaphoreType.DMA] * _NBUF,
        compiler_params=_SC_LINEAR,
    )(table_flat, src_sh, dst_sh)


_GRID = N_PAD // 128
DHH = DH // 2


def _diag(v_row):
    ir = lax.broadcasted_iota(jnp.int32, (128, 128), 0)
    ic = lax.broadcasted_iota(jnp.int32, (128, 128), 1)
    return jnp.where(ir == ic, jnp.broadcast_to(v_row, (128, 128)), 0.0)


BLK = 1024
Q = BLK // 128
_TGRID = N_PAD // BLK


def _tc1_body(degp_ref, x_ref, w1_ref, t1_ref, dinv_ref):
    y = jnp.dot(x_ref[...], w1_ref[...], preferred_element_type=jnp.float32)
    for q in range(Q):
        deg = degp_ref[0, q] + degp_ref[1, q] + 1.0
        dinv = lax.rsqrt(jnp.maximum(deg, 1.0))
        dinv_ref[q] = dinv
        yq = jnp.dot(_diag(dinv), y[128 * q:128 * (q + 1)],
                     preferred_element_type=jnp.float32)
        t1_ref[0, 128 * q:128 * (q + 1)] = yq[:, :DHH]
        t1_ref[1, 128 * q:128 * (q + 1)] = yq[:, DHH:]


def _tc_scale_in(degp, x_pad, w1):
    return pl.pallas_call(
        _tc1_body,
        grid=(_TGRID,),
        in_specs=[
            pl.BlockSpec((NC, Q, 1, 128), lambda i: (0, i, 0, 0)),
            pl.BlockSpec((BLK, DI), lambda i: (i, 0)),
            pl.BlockSpec((DI, DH), lambda i: (0, 0)),
        ],
        out_specs=[
            pl.BlockSpec((NC, BLK, DHH), lambda i: (0, i, 0)),
            pl.BlockSpec((Q, 1, 128), lambda i: (i, 0, 0)),
        ],
        out_shape=[
            jax.ShapeDtypeStruct((NC, N_PAD, DHH), jnp.float32),
            jax.ShapeDtypeStruct((_GRID, 1, 128), jnp.float32),
        ],
    )(degp, x_pad, w1)


def _tc2_body(pp_ref, t1_ref, dinv_ref, b1_ref, w2_ref, t2_ref):
    agg = jnp.concatenate([pp_ref[0], pp_ref[1]], axis=-1)
    t1 = jnp.concatenate([t1_ref[0], t1_ref[1]], axis=-1)
    u = agg + t1
    for q in range(Q):
        dmat = _diag(dinv_ref[q])
        pre = jnp.dot(dmat, u[128 * q:128 * (q + 1)],
                      preferred_element_type=jnp.float32)
        h = jnp.maximum(pre + b1_ref[...], 0.0)
        y2 = jnp.dot(h, w2_ref[...], preferred_element_type=jnp.float32)
        y2 = jnp.dot(dmat, y2, preferred_element_type=jnp.float32)
        t2_ref[0, 128 * q:128 * (q + 1)] = y2[:, :DOH]
        t2_ref[1, 128 * q:128 * (q + 1)] = y2[:, DOH:]


def _tc_hidden(pp1, table1, dinv, b1r, w2p):
    return pl.pallas_call(
        _tc2_body,
        grid=(_TGRID,),
        in_specs=[
            pl.BlockSpec((NC, BLK, DHH), lambda i: (0, i, 0)),
            pl.BlockSpec((NC, BLK, DHH), lambda i: (0, i, 0)),
            pl.BlockSpec((Q, 1, 128), lambda i: (i, 0, 0)),
            pl.BlockSpec((1, DH), lambda i: (0, 0)),
            pl.BlockSpec((DH, DOP), lambda i: (0, 0)),
        ],
        out_specs=pl.BlockSpec((NC, BLK, DOH), lambda i: (0, i, 0)),
        out_shape=jax.ShapeDtypeStruct((NC, N_PAD, DOH), jnp.float32),
    )(pp1, table1, dinv, b1r, w2p)


def _tc3_body(pp_ref, t2_ref, dinv_ref, b2_ref, out_ref):
    agg = jnp.concatenate([pp_ref[0], pp_ref[1]], axis=-1)
    t2 = jnp.concatenate([t2_ref[0], t2_ref[1]], axis=-1)
    u = agg + t2
    for q in range(Q):
        z = jnp.dot(_diag(dinv_ref[q]), u[128 * q:128 * (q + 1)],
                    preferred_element_type=jnp.float32)
        z = z + b2_ref[...]
        m = jnp.max(z, axis=1, keepdims=True)
        e = jnp.exp(z - m)
        ssum = jnp.sum(e, axis=1, keepdims=True)
        out_ref[128 * q:128 * (q + 1)] = (z - m) - jnp.log(ssum)


def _tc_logsoftmax(pp2, table2, dinv, b2r):
    return pl.pallas_call(
        _tc3_body,
        grid=(_TGRID,),
        in_specs=[
            pl.BlockSpec((NC, BLK, DOH), lambda i: (0, i, 0)),
            pl.BlockSpec((NC, BLK, DOH), lambda i: (0, i, 0)),
            pl.BlockSpec((Q, 1, 128), lambda i: (i, 0, 0)),
            pl.BlockSpec((1, DOP), lambda i: (0, 0)),
        ],
        out_specs=pl.BlockSpec((BLK, DOP), lambda i: (i, 0)),
        out_shape=jax.ShapeDtypeStruct((N_PAD, DOP), jnp.float32),
    )(pp2, table2, dinv, b2r)


def kernel(x, edge_index, W1, b1, W2, b2):
    pad_n = E_PAD - EE
    pad_rows = jnp.asarray(NN + (np.arange(pad_n, dtype=np.int32)
                                 % (N_PAD - NN)))
    src_flat = jnp.concatenate([edge_index[0], pad_rows])
    dst_flat = jnp.concatenate([edge_index[1], pad_rows])
    dst_e = dst_flat.reshape(NW, CWE, CH)
    src_c = jnp.stack([src_flat, src_flat + N_PAD]).reshape(NC, NS, CWC, CH)
    dst_c = dst_flat.reshape(NS, CWC, CH)
    x_pad = jnp.zeros((N_PAD, DI), jnp.float32).at[:NN].set(x)
    w2p = jnp.zeros((DH, DOP), jnp.float32).at[:, :DO].set(W2)
    b1r = b1.reshape(1, DH)
    b2r = jnp.full((1, DOP), -1e30, jnp.float32).at[0, :DO].set(b2)

    degp = _sc_degree(dst_e).reshape(NC, _GRID, 1, 128)
    table1, dinv = _tc_scale_in(degp, x_pad, W1)
    pp1 = _sc_aggregate_cols(table1.reshape(NC * N_PAD, DHH), src_c, dst_c,
                             DHH).reshape(NC, N_PAD, DHH)
    table2 = _tc_hidden(pp1, table1, dinv, b1r, w2p)
    pp2 = _sc_aggregate_cols(table2.reshape(NC * N_PAD, DOH), src_c, dst_c,
                             DOH).reshape(NC, N_PAD, DOH)
    out = _tc_logsoftmax(pp2, table2, dinv, b2r)
    return out[:NN, :DO]


# speedup vs baseline: 1.5391x; 1.0675x over previous
import jax
import jax.numpy as jnp
import numpy as np
from jax import lax
from jax.experimental import pallas as pl
from jax.experimental.pallas import tpu as pltpu
from jax.experimental.pallas import tpu_sc as plsc

NN = 10000
EE = 320000
DI = 128
DH = 128
DO = 40
DOP = 64

NC = 2
NS = 16
NW = NC * NS
N_PAD = 10240
E_PAD = 327680
CH = 128
CWE = E_PAD // NW // CH
CWC = E_PAD // NS // CH
RPS = N_PAD // NS
_NBUF = 4

_mesh = plsc.VectorSubcoreMesh(core_axis_name="c", subcore_axis_name="s")
_SC_LINEAR = pltpu.CompilerParams(use_tc_tiling_on_sc=False)


def _deg_body(dst_hbm, degp_hbm, dst_idx, ones_v, zbuf, acc, sem):
    c = lax.axis_index("c")
    s = lax.axis_index("s")
    wid = s * NC + c
    pltpu.sync_copy(dst_hbm.at[wid], dst_idx)
    for k in range(CH // 16):
        ones_v[pl.ds(16 * k, 16)] = jnp.ones((16,), jnp.float32)
    for k in range(RPS // 16):
        zbuf[pl.ds(16 * k, 16)] = jnp.zeros((16,), jnp.float32)
    pltpu.sync_copy(zbuf, acc.at[pl.ds(s * RPS, RPS)])
    plsc.subcore_barrier()

    def body(j, carry):
        pltpu.sync_copy(ones_v, acc.at[dst_idx.at[j]], add=True)
        return carry

    lax.fori_loop(0, CWE, body, 0)
    plsc.subcore_barrier()
    pltpu.sync_copy(acc.at[pl.ds(s * RPS, RPS)],
                    degp_hbm.at[pl.ds(c * N_PAD + s * RPS, RPS)])
    del sem


def _sc_degree(dst_sh):
    return pl.kernel(
        _deg_body,
        out_type=jax.ShapeDtypeStruct((NC * N_PAD,), jnp.float32),
        mesh=_mesh,
        scratch_types=[
            pltpu.VMEM((CWE, CH), jnp.int32),
            pltpu.VMEM((CH,), jnp.float32),
            pltpu.VMEM((RPS,), jnp.float32),
            pltpu.VMEM_SHARED((N_PAD,), jnp.float32),
            pltpu.SemaphoreType.DMA,
        ],
        compiler_params=_SC_LINEAR,
    )(dst_sh)


def _agg_pipeline(table_hbm, src_idx, dst_idx, rows, acc, sems, n_chunks):
    for k in range(_NBUF - 1):
        pltpu.async_copy(table_hbm.at[src_idx.at[k]], rows.at[k], sems[k])

    def body(jj, carry):
        for k in range(_NBUF):
            j = _NBUF * jj + k
            pltpu.make_async_copy(
                table_hbm.at[src_idx.at[j]], rows.at[k], sems[k]).wait()
            pltpu.sync_copy(rows.at[k], acc.at[dst_idx.at[j]], add=True)
            kn = (k + _NBUF - 1) % _NBUF

            @pl.when(j + _NBUF - 1 < n_chunks)
            def _():
                pltpu.async_copy(table_hbm.at[src_idx.at[j + _NBUF - 1]],
                                 rows.at[kn], sems[kn])
        return carry

    lax.fori_loop(0, n_chunks // _NBUF, body, 0)


DOH = DOP // 2


def _make_agg_c_body(dc):
    def _agg_c_body(table_hbm, src_hbm, dst_hbm, out_hbm,
                    src_idx, dst_idx, rows, zbuf, acc, *sems):
        c = lax.axis_index("c")
        s = lax.axis_index("s")
        pltpu.sync_copy(src_hbm.at[c, s], src_idx)
        pltpu.sync_copy(dst_hbm.at[s], dst_idx)
        for r in range(16):
            for k in range(dc // 16):
                zbuf[r, pl.ds(16 * k, 16)] = jnp.zeros((16,), jnp.float32)
        for i in range(RPS // 16):
            pltpu.sync_copy(zbuf, acc.at[pl.ds(s * RPS + i * 16, 16)])
        plsc.subcore_barrier()
        _agg_pipeline(table_hbm, src_idx, dst_idx, rows, acc, sems, CWC)
        plsc.subcore_barrier()
        pltpu.sync_copy(acc.at[pl.ds(s * RPS, RPS)],
                        out_hbm.at[pl.ds(c * N_PAD + s * RPS, RPS)])

    return _agg_c_body


def _sc_aggregate_cols(table_flat, src_sh, dst_sh, dc):
    return pl.kernel(
        _make_agg_c_body(dc),
        out_type=jax.ShapeDtypeStruct((NC * N_PAD, dc), jnp.float32),
        mesh=_mesh,
        scratch_types=[
            pltpu.VMEM((CWC, CH), jnp.int32),
            pltpu.VMEM((CWC, CH), jnp.int32),
            pltpu.VMEM((_NBUF, CH, dc), jnp.float32),
            pltpu.VMEM((16, dc), jnp.float32),
            pltpu.VMEM_SHARED((N_PAD, dc), jnp.float32),
        ] + [pltpu.SemaphoreType.DMA] * _NBUF,
        compiler_params=_SC_LINEAR,
    )(table_flat, src_sh, dst_sh)


def _agg_e_body(table_hbm, src_hbm, dst_hbm, out_hbm,
                src_idx, dst_idx, rows, zbuf, acc, *sems):
    c = lax.axis_index("c")
    s = lax.axis_index("s")
    wid = s * NC + c
    pltpu.sync_copy(src_hbm.at[wid], src_idx)
    pltpu.sync_copy(dst_hbm.at[wid], dst_idx)
    for r in range(16):
        for k in range(DOP // 16):
            zbuf[r, pl.ds(16 * k, 16)] = jnp.zeros((16,), jnp.float32)
    for i in range(RPS // 16):
        pltpu.sync_copy(zbuf, acc.at[pl.ds(s * RPS + i * 16, 16)])
    plsc.subcore_barrier()
    _agg_pipeline(table_hbm, src_idx, dst_idx, rows, acc, sems, CWE)
    plsc.subcore_barrier()
    pltpu.sync_copy(acc.at[pl.ds(s * RPS, RPS)],
                    out_hbm.at[pl.ds(c * N_PAD + s * RPS, RPS)])


def _sc_aggregate_edges(table, src_sh, dst_sh):
    return pl.kernel(
        _agg_e_body,
        out_type=jax.ShapeDtypeStruct((NC * N_PAD, DOP), jnp.float32),
        mesh=_mesh,
        scratch_types=[
            pltpu.VMEM((CWE, CH), jnp.int32),
            pltpu.VMEM((CWE, CH), jnp.int32),
            pltpu.VMEM((_NBUF, CH, DOP), jnp.float32),
            pltpu.VMEM((16, DOP), jnp.float32),
            pltpu.VMEM_SHARED((N_PAD, DOP), jnp.float32),
        ] + [pltpu.SemaphoreType.DMA] * _NBUF,
        compiler_params=_SC_LINEAR,
    )(table, src_sh, dst_sh)


_GRID = N_PAD // 128
DHH = DH // 2


def _diag(v_row):
    ir = lax.broadcasted_iota(jnp.int32, (128, 128), 0)
    ic = lax.broadcasted_iota(jnp.int32, (128, 128), 1)
    return jnp.where(ir == ic, jnp.broadcast_to(v_row, (128, 128)), 0.0)


BLK = 1024
Q = BLK // 128
_TGRID = N_PAD // BLK


def _tc1_body(degp_ref, x_ref, w1_ref, t1_ref, dinv_ref):
    y = jnp.dot(x_ref[...], w1_ref[...], preferred_element_type=jnp.float32)
    for q in range(Q):
        deg = degp_ref[0, q] + degp_ref[1, q] + 1.0
        dinv = lax.rsqrt(jnp.maximum(deg, 1.0))
        dinv_ref[q] = dinv
        yq = jnp.dot(_diag(dinv), y[128 * q:128 * (q + 1)],
                     preferred_element_type=jnp.float32)
        t1_ref[0, 128 * q:128 * (q + 1)] = yq[:, :DHH]
        t1_ref[1, 128 * q:128 * (q + 1)] = yq[:, DHH:]


def _tc_scale_in(degp, x_pad, w1):
    return pl.pallas_call(
        _tc1_body,
        grid=(_TGRID,),
        in_specs=[
            pl.BlockSpec((NC, Q, 1, 128), lambda i: (0, i, 0, 0)),
            pl.BlockSpec((BLK, DI), lambda i: (i, 0)),
            pl.BlockSpec((DI, DH), lambda i: (0, 0)),
        ],
        out_specs=[
            pl.BlockSpec((NC, BLK, DHH), lambda i: (0, i, 0)),
            pl.BlockSpec((Q, 1, 128), lambda i: (i, 0, 0)),
        ],
        out_shape=[
            jax.ShapeDtypeStruct((NC, N_PAD, DHH), jnp.float32),
            jax.ShapeDtypeStruct((_GRID, 1, 128), jnp.float32),
        ],
    )(degp, x_pad, w1)


def _tc2_body(pp_ref, t1_ref, dinv_ref, b1_ref, w2_ref, t2_ref):
    agg = jnp.concatenate([pp_ref[0], pp_ref[1]], axis=-1)
    t1 = jnp.concatenate([t1_ref[0], t1_ref[1]], axis=-1)
    u = agg + t1
    for q in range(Q):
        dmat = _diag(dinv_ref[q])
        pre = jnp.dot(dmat, u[128 * q:128 * (q + 1)],
                      preferred_element_type=jnp.float32)
        h = jnp.maximum(pre + b1_ref[...], 0.0)
        y2 = jnp.dot(h, w2_ref[...], preferred_element_type=jnp.float32)
        y2 = jnp.dot(dmat, y2, preferred_element_type=jnp.float32)
        t2_ref[128 * q:128 * (q + 1)] = y2


def _tc_hidden(pp1, table1, dinv, b1r, w2p):
    return pl.pallas_call(
        _tc2_body,
        grid=(_TGRID,),
        in_specs=[
            pl.BlockSpec((NC, BLK, DHH), lambda i: (0, i, 0)),
            pl.BlockSpec((NC, BLK, DHH), lambda i: (0, i, 0)),
            pl.BlockSpec((Q, 1, 128), lambda i: (i, 0, 0)),
            pl.BlockSpec((1, DH), lambda i: (0, 0)),
            pl.BlockSpec((DH, DOP), lambda i: (0, 0)),
        ],
        out_specs=pl.BlockSpec((BLK, DOP), lambda i: (i, 0)),
        out_shape=jax.ShapeDtypeStruct((N_PAD, DOP), jnp.float32),
    )(pp1, table1, dinv, b1r, w2p)


def _tc3_body(pp_ref, t2_ref, dinv_ref, b2_ref, out_ref):
    u = pp_ref[0] + pp_ref[1] + t2_ref[...]
    for q in range(Q):
        z = jnp.dot(_diag(dinv_ref[q]), u[128 * q:128 * (q + 1)],
                    preferred_element_type=jnp.float32)
        z = z + b2_ref[...]
        m = jnp.max(z, axis=1, keepdims=True)
        e = jnp.exp(z - m)
        ssum = jnp.sum(e, axis=1, keepdims=True)
        out_ref[128 * q:128 * (q + 1)] = (z - m) - jnp.log(ssum)


def _tc_logsoftmax(pp2, table2, dinv, b2r):
    return pl.pallas_call(
        _tc3_body,
        grid=(_TGRID,),
        in_specs=[
            pl.BlockSpec((NC, BLK, DOP), lambda i: (0, i, 0)),
            pl.BlockSpec((BLK, DOP), lambda i: (i, 0)),
            pl.BlockSpec((Q, 1, 128), lambda i: (i, 0, 0)),
            pl.BlockSpec((1, DOP), lambda i: (0, 0)),
        ],
        out_specs=pl.BlockSpec((BLK, DOP), lambda i: (i, 0)),
        out_shape=jax.ShapeDtypeStruct((N_PAD, DOP), jnp.float32),
    )(pp2, table2, dinv, b2r)


def kernel(x, edge_index, W1, b1, W2, b2):
    pad_n = E_PAD - EE
    pad_rows = jnp.asarray(NN + (np.arange(pad_n, dtype=np.int32)
                                 % (N_PAD - NN)))
    src_flat = jnp.concatenate([edge_index[0], pad_rows])
    dst_flat = jnp.concatenate([edge_index[1], pad_rows])
    src_e = src_flat.reshape(NW, CWE, CH)
    dst_e = dst_flat.reshape(NW, CWE, CH)
    src_c = jnp.stack([src_flat, src_flat + N_PAD]).reshape(NC, NS, CWC, CH)
    dst_c = dst_flat.reshape(NS, CWC, CH)
    x_pad = jnp.zeros((N_PAD, DI), jnp.float32).at[:NN].set(x)
    w2p = jnp.zeros((DH, DOP), jnp.float32).at[:, :DO].set(W2)
    b1r = b1.reshape(1, DH)
    b2r = jnp.full((1, DOP), -1e30, jnp.float32).at[0, :DO].set(b2)

    degp = _sc_degree(dst_e).reshape(NC, _GRID, 1, 128)
    table1, dinv = _tc_scale_in(degp, x_pad, W1)
    pp1 = _sc_aggregate_cols(table1.reshape(NC * N_PAD, DHH), src_c, dst_c,
                             DHH).reshape(NC, N_PAD, DHH)
    table2 = _tc_hidden(pp1, table1, dinv, b1r, w2p)
    pp2 = _sc_aggregate_edges(table2, src_e, dst_e).reshape(NC, N_PAD, DOP)
    out = _tc_logsoftmax(pp2, table2, dinv, b2r)
    return out[:NN, :DO]
